# baseline (speedup 1.0000x reference)
import functools

import jax
import jax.numpy as jnp
from jax import lax
from jax.experimental import pallas as pl
from jax.experimental.pallas import tpu as pltpu
from jax.experimental.pallas import tpu_sc as plsc

BATCH = 16384
EMB = 32
ROWS = 1000000
LANES = 16
NUM_WORKERS = 32
BPW = BATCH // NUM_WORKERS
NCHUNK = 4
CHUNK = BPW // NCHUNK
GROUPS = BPW // LANES
GPC = CHUNK // LANES
NB = 4
BPITCH = 129
SPITCH = 513


def _mf_body(user_hbm, item_hbm, pt_hbm, qt_hbm, out_hbm,
             uidx, iidx, ublk, iblk, stage, outv, *sems):
    wid = lax.axis_index("s") * 2 + lax.axis_index("c")
    base = wid * BPW

    for j in range(NCHUNK):
        pltpu.sync_copy(user_hbm.at[pl.ds(base + j * CHUNK, CHUNK)], uidx.at[j])
        pltpu.sync_copy(item_hbm.at[pl.ds(base + j * CHUNK, CHUNK)], iidx.at[j])

    dvec0 = lax.iota(jnp.int32, LANES)
    dvec1 = dvec0 + LANES

    def _load_group(g):
        j = g // GPC
        o = (g % GPC) * LANES
        return uidx[j, pl.ds(o, LANES)], iidx[j, pl.ds(o, LANES)]

    def _fire(ru, ri, b):
        cu = pl.multiple_of((ru >> 7) << 7, 128)
        ci = pl.multiple_of((ri >> 7) << 7, 128)
        pltpu.async_copy(pt_hbm.at[:, pl.ds(cu, 128)],
                         ublk.at[b, :, pl.ds(0, 128)], sems[b])
        pltpu.async_copy(qt_hbm.at[:, pl.ds(ci, 128)],
                         iblk.at[b, :, pl.ds(0, 128)], sems[b])

    def _drain(b):
        pltpu.make_async_copy(pt_hbm.at[:, pl.ds(0, 128)],
                              ublk.at[b, :, pl.ds(0, 128)], sems[b]).wait()
        pltpu.make_async_copy(qt_hbm.at[:, pl.ds(0, 128)],
                              iblk.at[b, :, pl.ds(0, 128)], sems[b]).wait()

    def _consume(ru, ri, col, b):
        lu = jnp.full((LANES,), ru & 127, jnp.int32)
        li = jnp.full((LANES,), ri & 127, jnp.int32)
        bvec = jnp.full((LANES,), b, jnp.int32)
        u0 = plsc.load_gather(ublk, [bvec, dvec0, lu])
        u1 = plsc.load_gather(ublk, [bvec, dvec1, lu])
        i0 = plsc.load_gather(iblk, [bvec, dvec0, li])
        i1 = plsc.load_gather(iblk, [bvec, dvec1, li])
        part = u0 * i0 + u1 * i1
        plsc.store_scatter(stage, [dvec0, jnp.full((LANES,), col, jnp.int32)],
                           part)

    u16_0, w16_0 = _load_group(0)
    for b in range(NB):
        _fire(u16_0[b], w16_0[b], b)

    def loop_body(g, carry):
        cur_u, cur_w = carry
        gn = jnp.minimum(g + 1, GROUPS - 1)
        nxt_u, nxt_w = _load_group(gn)
        for l in range(LANES):
            b = l % NB
            _drain(b)
            _consume(cur_u[l], cur_w[l], g * LANES + l, b)
            if l + NB < LANES:
                _fire(cur_u[l + NB], cur_w[l + NB], b)
            else:
                ln = l + NB - LANES

                @pl.when(g < GROUPS - 1)
                def _():
                    _fire(nxt_u[ln], nxt_w[ln], b)
        return nxt_u, nxt_w

    lax.fori_loop(0, GROUPS, loop_body, (u16_0, w16_0))

    def out_body(g, carry):
        s0 = g * LANES
        acc = stage[0, pl.ds(s0, LANES)]
        for j in range(1, LANES):
            acc = acc + stage[j, pl.ds(s0, LANES)]
        outv[pl.ds(s0, LANES)] = 1.0 / (1.0 + jnp.exp(-acc))
        return carry

    lax.fori_loop(0, GROUPS, out_body, 0)

    pltpu.sync_copy(outv, out_hbm.at[pl.ds(base, BPW)])


_mf_kernel = functools.partial(
    pl.kernel,
    mesh=plsc.VectorSubcoreMesh(core_axis_name="c", subcore_axis_name="s"),
    out_type=jax.ShapeDtypeStruct((BATCH,), jnp.float32),
    compiler_params=pltpu.CompilerParams(
        needs_layout_passes=False, use_tc_tiling_on_sc=True),
    scratch_types=[
        pltpu.VMEM((NCHUNK, CHUNK), jnp.int32),
        pltpu.VMEM((NCHUNK, CHUNK), jnp.int32),
        pltpu.VMEM((NB, EMB, BPITCH), jnp.float32),
        pltpu.VMEM((NB, EMB, BPITCH), jnp.float32),
        pltpu.VMEM((LANES, SPITCH), jnp.float32),
        pltpu.VMEM((BPW,), jnp.float32),
    ] + [pltpu.SemaphoreType.DMA] * NB,
)(_mf_body)


@jax.jit
def kernel(x, P, Q):
    x = x.astype(jnp.int32)
    user = x[:, 0]
    item = x[:, 1]
    return _mf_kernel(user, item, P.T, Q.T)


# speedup vs baseline: 1.1030x; 1.1030x over previous
import functools

import jax
import jax.numpy as jnp
from jax import lax
from jax.experimental import pallas as pl
from jax.experimental.pallas import tpu as pltpu
from jax.experimental.pallas import tpu_sc as plsc

BATCH = 16384
EMB = 32
ROWS = 1000000
LANES = 16
NUM_WORKERS = 32
BPW = BATCH // NUM_WORKERS
NCHUNK = 4
CHUNK = BPW // NCHUNK
GROUPS = BPW // LANES
GPC = CHUNK // LANES
NB = 6
BPITCH = 129
SPITCH = 513


def _mf_body(user_hbm, item_hbm, pt_hbm, qt_hbm, out_hbm,
             uidx, iidx, ublk, iblk, stage, outv, sems):
    wid = lax.axis_index("s") * 2 + lax.axis_index("c")
    base = wid * BPW

    for j in range(NCHUNK):
        pltpu.sync_copy(user_hbm.at[pl.ds(base + j * CHUNK, CHUNK)], uidx.at[j])
        pltpu.sync_copy(item_hbm.at[pl.ds(base + j * CHUNK, CHUNK)], iidx.at[j])

    dvec0 = lax.iota(jnp.int32, LANES)
    dvec1 = dvec0 + LANES

    def _load_group(g):
        j = g // GPC
        o = (g % GPC) * LANES
        return uidx[j, pl.ds(o, LANES)], iidx[j, pl.ds(o, LANES)]

    def _fire(ru, ri, b):
        cu = pl.multiple_of((ru >> 7) << 7, 128)
        ci = pl.multiple_of((ri >> 7) << 7, 128)
        pltpu.async_copy(pt_hbm.at[:, pl.ds(cu, 128)],
                         ublk.at[b, :, pl.ds(0, 128)], sems.at[b])
        pltpu.async_copy(qt_hbm.at[:, pl.ds(ci, 128)],
                         iblk.at[b, :, pl.ds(0, 128)], sems.at[b])

    def _drain(b):
        pltpu.make_async_copy(pt_hbm.at[:, pl.ds(0, 128)],
                              ublk.at[b, :, pl.ds(0, 128)], sems.at[b]).wait()
        pltpu.make_async_copy(qt_hbm.at[:, pl.ds(0, 128)],
                              iblk.at[b, :, pl.ds(0, 128)], sems.at[b]).wait()

    def _consume(ru, ri, col, b):
        lu = jnp.full((LANES,), ru & 127, jnp.int32)
        li = jnp.full((LANES,), ri & 127, jnp.int32)
        bvec = jnp.full((LANES,), b, jnp.int32)
        u0 = plsc.load_gather(ublk, [bvec, dvec0, lu])
        u1 = plsc.load_gather(ublk, [bvec, dvec1, lu])
        i0 = plsc.load_gather(iblk, [bvec, dvec0, li])
        i1 = plsc.load_gather(iblk, [bvec, dvec1, li])
        part = u0 * i0 + u1 * i1
        plsc.store_scatter(stage, [dvec0, jnp.full((LANES,), col, jnp.int32)],
                           part)

    u16_0, w16_0 = _load_group(0)
    for b in range(NB):
        _fire(u16_0[b], w16_0[b], b)

    def loop_body(g, carry):
        cur_u, cur_w = carry
        gn = jnp.minimum(g + 1, GROUPS - 1)
        nxt_u, nxt_w = _load_group(gn)
        for l in range(LANES):
            b = lax.rem((LANES % NB) * g + l, NB)
            _drain(b)
            _consume(cur_u[l], cur_w[l], g * LANES + l, b)
            if l + NB < LANES:
                _fire(cur_u[l + NB], cur_w[l + NB], b)
            else:
                ln = l + NB - LANES

                @pl.when(g < GROUPS - 1)
                def _():
                    _fire(nxt_u[ln], nxt_w[ln], b)
        return nxt_u, nxt_w

    lax.fori_loop(0, GROUPS, loop_body, (u16_0, w16_0))

    def out_body(g, carry):
        s0 = g * LANES
        acc = stage[0, pl.ds(s0, LANES)]
        for j in range(1, LANES):
            acc = acc + stage[j, pl.ds(s0, LANES)]
        outv[pl.ds(s0, LANES)] = 1.0 / (1.0 + jnp.exp(-acc))
        return carry

    lax.fori_loop(0, GROUPS, out_body, 0)

    pltpu.sync_copy(outv, out_hbm.at[pl.ds(base, BPW)])


_mf_kernel = functools.partial(
    pl.kernel,
    mesh=plsc.VectorSubcoreMesh(core_axis_name="c", subcore_axis_name="s"),
    out_type=jax.ShapeDtypeStruct((BATCH,), jnp.float32),
    compiler_params=pltpu.CompilerParams(
        needs_layout_passes=False, use_tc_tiling_on_sc=True),
    scratch_types=[
        pltpu.VMEM((NCHUNK, CHUNK), jnp.int32),
        pltpu.VMEM((NCHUNK, CHUNK), jnp.int32),
        pltpu.VMEM((NB, EMB, BPITCH), jnp.float32),
        pltpu.VMEM((NB, EMB, BPITCH), jnp.float32),
        pltpu.VMEM((LANES, SPITCH), jnp.float32),
        pltpu.VMEM((BPW,), jnp.float32),
        pltpu.SemaphoreType.DMA((NB,)),
    ],
)(_mf_body)


@jax.jit
def kernel(x, P, Q):
    x = x.astype(jnp.int32)
    user = x[:, 0]
    item = x[:, 1]
    return _mf_kernel(user, item, P.T, Q.T)


# speedup vs baseline: 1.1417x; 1.0351x over previous
import functools

import jax
import jax.numpy as jnp
from jax import lax
from jax.experimental import pallas as pl
from jax.experimental.pallas import tpu as pltpu
from jax.experimental.pallas import tpu_sc as plsc

BATCH = 16384
EMB = 32
ROWS = 1000000
LANES = 16
NUM_WORKERS = 32
BPW = BATCH // NUM_WORKERS
NCHUNK = 4
CHUNK = BPW // NCHUNK
GROUPS = BPW // LANES
GPC = CHUNK // LANES
NB = 7
BPITCH = 129
SPITCH = 513


def _mf_body(user_hbm, item_hbm, pt_hbm, qt_hbm, out_hbm,
             uidx, iidx, ublk, iblk, stage, outv, sems):
    wid = lax.axis_index("s") * 2 + lax.axis_index("c")
    base = wid * BPW

    for j in range(NCHUNK):
        pltpu.sync_copy(user_hbm.at[pl.ds(base + j * CHUNK, CHUNK)], uidx.at[j])
        pltpu.sync_copy(item_hbm.at[pl.ds(base + j * CHUNK, CHUNK)], iidx.at[j])

    dvec0 = lax.iota(jnp.int32, LANES)
    dvec1 = dvec0 + LANES

    def _load_group(g):
        j = g // GPC
        o = (g % GPC) * LANES
        return uidx[j, pl.ds(o, LANES)], iidx[j, pl.ds(o, LANES)]

    def _fire(ru, ri, b):
        cu = pl.multiple_of((ru >> 7) << 7, 128)
        ci = pl.multiple_of((ri >> 7) << 7, 128)
        pltpu.async_copy(pt_hbm.at[:, pl.ds(cu, 128)],
                         ublk.at[b, :, pl.ds(0, 128)], sems.at[b])
        pltpu.async_copy(qt_hbm.at[:, pl.ds(ci, 128)],
                         iblk.at[b, :, pl.ds(0, 128)], sems.at[b])

    def _drain(b):
        pltpu.make_async_copy(pt_hbm.at[:, pl.ds(0, 128)],
                              ublk.at[b, :, pl.ds(0, 128)], sems.at[b]).wait()
        pltpu.make_async_copy(qt_hbm.at[:, pl.ds(0, 128)],
                              iblk.at[b, :, pl.ds(0, 128)], sems.at[b]).wait()

    def _consume(ru, ri, col, b):
        lu = jnp.full((LANES,), ru & 127, jnp.int32)
        li = jnp.full((LANES,), ri & 127, jnp.int32)
        bvec = jnp.full((LANES,), b, jnp.int32)
        u0 = plsc.load_gather(ublk, [bvec, dvec0, lu])
        u1 = plsc.load_gather(ublk, [bvec, dvec1, lu])
        i0 = plsc.load_gather(iblk, [bvec, dvec0, li])
        i1 = plsc.load_gather(iblk, [bvec, dvec1, li])
        part = u0 * i0 + u1 * i1
        plsc.store_scatter(stage, [dvec0, jnp.full((LANES,), col, jnp.int32)],
                           part)

    u16_0, w16_0 = _load_group(0)
    for b in range(NB):
        _fire(u16_0[b], w16_0[b], b)

    def loop_body(g, carry):
        cur_u, cur_w = carry
        gn = jnp.minimum(g + 1, GROUPS - 1)
        nxt_u, nxt_w = _load_group(gn)
        for l in range(LANES):
            b = lax.rem((LANES % NB) * g + l, NB)
            _drain(b)
            _consume(cur_u[l], cur_w[l], g * LANES + l, b)
            if l + NB < LANES:
                _fire(cur_u[l + NB], cur_w[l + NB], b)
            else:
                ln = l + NB - LANES

                @pl.when(g < GROUPS - 1)
                def _():
                    _fire(nxt_u[ln], nxt_w[ln], b)
        return nxt_u, nxt_w

    lax.fori_loop(0, GROUPS, loop_body, (u16_0, w16_0))

    def out_body(g, carry):
        s0 = g * LANES
        acc = stage[0, pl.ds(s0, LANES)]
        for j in range(1, LANES):
            acc = acc + stage[j, pl.ds(s0, LANES)]
        outv[pl.ds(s0, LANES)] = 1.0 / (1.0 + jnp.exp(-acc))
        return carry

    lax.fori_loop(0, GROUPS, out_body, 0)

    pltpu.sync_copy(outv, out_hbm.at[pl.ds(base, BPW)])


_mf_kernel = functools.partial(
    pl.kernel,
    mesh=plsc.VectorSubcoreMesh(core_axis_name="c", subcore_axis_name="s"),
    out_type=jax.ShapeDtypeStruct((BATCH,), jnp.float32),
    compiler_params=pltpu.CompilerParams(
        needs_layout_passes=False, use_tc_tiling_on_sc=True),
    scratch_types=[
        pltpu.VMEM((NCHUNK, CHUNK), jnp.int32),
        pltpu.VMEM((NCHUNK, CHUNK), jnp.int32),
        pltpu.VMEM((NB, EMB, BPITCH), jnp.float32),
        pltpu.VMEM((NB, EMB, BPITCH), jnp.float32),
        pltpu.VMEM((LANES, SPITCH), jnp.float32),
        pltpu.VMEM((BPW,), jnp.float32),
        pltpu.SemaphoreType.DMA((NB,)),
    ],
)(_mf_body)


@jax.jit
def kernel(x, P, Q):
    x = x.astype(jnp.int32)
    user = x[:, 0]
    item = x[:, 1]
    return _mf_kernel(user, item, P.T, Q.T)


# speedup vs baseline: 1.1790x; 1.0327x over previous
import functools

import jax
import jax.numpy as jnp
from jax import lax
from jax.experimental import pallas as pl
from jax.experimental.pallas import tpu as pltpu
from jax.experimental.pallas import tpu_sc as plsc

BATCH = 16384
EMB = 32
ROWS = 1000000
LANES = 16
NUM_WORKERS = 32
BPW = BATCH // NUM_WORKERS
NCHUNK = 4
CHUNK = BPW // NCHUNK
GROUPS = BPW // LANES
GPC = CHUNK // LANES
NB = 8
BPITCH = 128
SPITCH = 513


def _mf_body(user_hbm, item_hbm, pt_hbm, qt_hbm, out_hbm,
             uidx, iidx, ublk, iblk, stage, outv, sems):
    wid = lax.axis_index("s") * 2 + lax.axis_index("c")
    base = wid * BPW

    for j in range(NCHUNK):
        pltpu.sync_copy(user_hbm.at[pl.ds(base + j * CHUNK, CHUNK)], uidx.at[j])
        pltpu.sync_copy(item_hbm.at[pl.ds(base + j * CHUNK, CHUNK)], iidx.at[j])

    dvec0 = lax.iota(jnp.int32, LANES)
    dvec1 = dvec0 + LANES

    def _load_group(g):
        j = g // GPC
        o = (g % GPC) * LANES
        return uidx[j, pl.ds(o, LANES)], iidx[j, pl.ds(o, LANES)]

    def _fire(ru, ri, b):
        cu = pl.multiple_of((ru >> 7) << 7, 128)
        ci = pl.multiple_of((ri >> 7) << 7, 128)
        pltpu.async_copy(pt_hbm.at[:, pl.ds(cu, 128)],
                         ublk.at[b, :, pl.ds(0, 128)], sems.at[b])
        pltpu.async_copy(qt_hbm.at[:, pl.ds(ci, 128)],
                         iblk.at[b, :, pl.ds(0, 128)], sems.at[b])

    def _drain(b):
        pltpu.make_async_copy(pt_hbm.at[:, pl.ds(0, 128)],
                              ublk.at[b, :, pl.ds(0, 128)], sems.at[b]).wait()
        pltpu.make_async_copy(qt_hbm.at[:, pl.ds(0, 128)],
                              iblk.at[b, :, pl.ds(0, 128)], sems.at[b]).wait()

    def _consume(ru, ri, col, b):
        lu = jnp.full((LANES,), ru & 127, jnp.int32)
        li = jnp.full((LANES,), ri & 127, jnp.int32)
        bvec = jnp.full((LANES,), b, jnp.int32)
        u0 = plsc.load_gather(ublk, [bvec, dvec0, lu])
        u1 = plsc.load_gather(ublk, [bvec, dvec1, lu])
        i0 = plsc.load_gather(iblk, [bvec, dvec0, li])
        i1 = plsc.load_gather(iblk, [bvec, dvec1, li])
        part = u0 * i0 + u1 * i1
        plsc.store_scatter(stage, [dvec0, jnp.full((LANES,), col, jnp.int32)],
                           part)

    u16_0, w16_0 = _load_group(0)
    for b in range(NB):
        _fire(u16_0[b], w16_0[b], b)

    def loop_body(g, carry):
        cur_u, cur_w = carry
        gn = jnp.minimum(g + 1, GROUPS - 1)
        nxt_u, nxt_w = _load_group(gn)
        for l in range(LANES):
            b = lax.rem((LANES % NB) * g + l, NB)
            _drain(b)
            _consume(cur_u[l], cur_w[l], g * LANES + l, b)
            if l + NB < LANES:
                _fire(cur_u[l + NB], cur_w[l + NB], b)
            else:
                ln = l + NB - LANES

                @pl.when(g < GROUPS - 1)
                def _():
                    _fire(nxt_u[ln], nxt_w[ln], b)
        return nxt_u, nxt_w

    lax.fori_loop(0, GROUPS, loop_body, (u16_0, w16_0))

    def out_body(g, carry):
        s0 = g * LANES
        acc = stage[0, pl.ds(s0, LANES)]
        for j in range(1, LANES):
            acc = acc + stage[j, pl.ds(s0, LANES)]
        outv[pl.ds(s0, LANES)] = 1.0 / (1.0 + jnp.exp(-acc))
        return carry

    lax.fori_loop(0, GROUPS, out_body, 0)

    pltpu.sync_copy(outv, out_hbm.at[pl.ds(base, BPW)])


_mf_kernel = functools.partial(
    pl.kernel,
    mesh=plsc.VectorSubcoreMesh(core_axis_name="c", subcore_axis_name="s"),
    out_type=jax.ShapeDtypeStruct((BATCH,), jnp.float32),
    compiler_params=pltpu.CompilerParams(
        needs_layout_passes=False, use_tc_tiling_on_sc=True),
    scratch_types=[
        pltpu.VMEM((NCHUNK, CHUNK), jnp.int32),
        pltpu.VMEM((NCHUNK, CHUNK), jnp.int32),
        pltpu.VMEM((NB, EMB, BPITCH), jnp.float32),
        pltpu.VMEM((NB, EMB, BPITCH), jnp.float32),
        pltpu.VMEM((LANES, SPITCH), jnp.float32),
        pltpu.VMEM((BPW,), jnp.float32),
        pltpu.SemaphoreType.DMA((NB,)),
    ],
)(_mf_body)


@jax.jit
def kernel(x, P, Q):
    x = x.astype(jnp.int32)
    user = x[:, 0]
    item = x[:, 1]
    return _mf_kernel(user, item, P.T, Q.T)
